# baseline (speedup 1.0000x reference)
import functools

import jax
import jax.numpy as jnp
from jax import lax
from jax.experimental import pallas as pl
from jax.experimental.pallas import tpu as pltpu
from jax.experimental.pallas import tpu_sc as plsc

_B, _V, _M, _R = 1024, 16, 8, 4096
_TINY = 1e-30

_info = plsc.get_sparse_core_info()
_NC, _NS, _L = _info.num_cores, _info.num_subcores, _info.num_lanes
_NW = _NC * _NS
_BPW = _B // _NW
_RC = 256

_R_SC = _R
_RBLK_TC = 256


def _sc_fire(mu_hbm, aT_hbm, out_hbm, mu_v, a_v, stage_v):
    n_rules = aT_hbm.shape[1]
    wid = lax.axis_index("s") * _NC + lax.axis_index("c")
    b0 = wid * _BPW
    pltpu.sync_copy(mu_hbm.at[pl.ds(b0 * _V * _M, _BPW * _V * _M)], mu_v)
    pltpu.sync_copy(aT_hbm, a_v)

    def rc_body(rc, _):
        c0 = rc * _RC

        def rb_body(rb, _):
            col = c0 + rb * _L
            idx = [a_v[v, pl.ds(col, _L)] + (v * _M) for v in range(_V)]

            def b_body(b, _):
                boff = b * (_V * _M)
                acc = plsc.load_gather(mu_v, [idx[0] + boff])
                for v in range(1, _V):
                    acc = acc * plsc.load_gather(mu_v, [idx[v] + boff])
                stage_v[b, pl.ds(rb * _L, _L)] = acc
                return 0

            lax.fori_loop(0, _BPW, b_body, 0)
            return 0

        lax.fori_loop(0, _RC // _L, rb_body, 0)
        pltpu.sync_copy(stage_v, out_hbm.at[pl.ds(b0, _BPW), pl.ds(c0, _RC)])
        return 0

    lax.fori_loop(0, n_rules // _RC, rc_body, 0)


def _sc_call(mu2d, aT):
    n_rules = aT.shape[1]
    f = functools.partial(
        pl.kernel,
        mesh=plsc.VectorSubcoreMesh(core_axis_name="c", subcore_axis_name="s"),
        out_type=jax.ShapeDtypeStruct((_B, n_rules), jnp.float32),
        compiler_params=pltpu.CompilerParams(needs_layout_passes=False),
        scratch_types=[
            pltpu.VMEM((_BPW * _V * _M,), jnp.float32),
            pltpu.VMEM((_V, n_rules), jnp.int32),
            pltpu.VMEM((_BPW, _RC), jnp.float32),
        ],
    )(_sc_fire)
    return f(jnp.reshape(mu2d, (_B * _V * _M,)), aT)


def _tc_fire_block(muT_ref, aT_ref, out_ref):
    lmu = jnp.log(jnp.maximum(muT_ref[...], _TINY))
    aT = aT_ref[...]
    a_tiled = jnp.concatenate([aT] * _M, axis=0)
    m_of_row = lax.broadcasted_iota(jnp.int32, (_M * _V, _RBLK_TC), 0) // _V
    oh = (a_tiled == m_of_row).astype(jnp.float32)
    acc = lax.dot_general(
        lmu, oh, (((1,), (0,)), ((), ())),
        precision=lax.Precision.HIGHEST,
        preferred_element_type=jnp.float32,
    )
    out_ref[...] = jnp.exp(acc)


def _tc_call(mu, aT):
    n_rules = aT.shape[1]
    muT = jnp.swapaxes(mu, 1, 2).reshape(_B, _M * _V)
    return pl.pallas_call(
        _tc_fire_block,
        grid=(n_rules // _RBLK_TC,),
        in_specs=[
            pl.BlockSpec((_B, _M * _V), lambda j: (0, 0)),
            pl.BlockSpec((_V, _RBLK_TC), lambda j: (0, j)),
        ],
        out_specs=pl.BlockSpec((_B, _RBLK_TC), lambda j: (0, j)),
        out_shape=jax.ShapeDtypeStruct((_B, n_rules), jnp.float32),
    )(muT, aT)


def kernel(mu, antecedents):
    batch_shape = mu.shape[:-2]
    mu = jnp.reshape(mu, (-1, _V, _M))
    aT = antecedents.T
    mu2d = jnp.reshape(mu, (_B, _V * _M))
    parts = []
    if _R_SC > 0:
        parts.append(_sc_call(mu2d, aT[:, :_R_SC]))
    if _R_SC < _R:
        parts.append(_tc_call(mu, aT[:, _R_SC:]))
    out = parts[0] if len(parts) == 1 else jnp.concatenate(parts, axis=1)
    return jnp.reshape(out, (*batch_shape, _R))
